# baseline (speedup 1.0000x reference)
import functools

import jax
import jax.numpy as jnp
from jax import lax
from jax.experimental import pallas as pl
from jax.experimental.pallas import tpu as pltpu
from jax.experimental.pallas import tpu_sc as plsc

N = 10000
NP = 10240
E = 320000
EP = 327680
CH = 128
NCH0 = 80
NCH1 = 80
STRIPE = NP // 16


def _sc_mesh():
    return plsc.VectorSubcoreMesh(core_axis_name="c", subcore_axis_name="s")


def _make_sc_scatter(F):
    @functools.partial(
        pl.kernel,
        mesh=_sc_mesh(),
        out_type=jax.ShapeDtypeStruct((2, NP, F), jnp.float32),
        scratch_types=[
            pltpu.VMEM((NCH1 + 1, CH), jnp.int32),
            pltpu.VMEM((8, CH), jnp.int32),
            pltpu.VMEM((CH, F), jnp.float32),
            pltpu.VMEM((CH, F), jnp.float32),
            pltpu.VMEM_SHARED((NP, F), jnp.float32),
            pltpu.SemaphoreType.DMA,
            pltpu.SemaphoreType.DMA,
        ],
    )
    def k(g_hbm, srcA_hbm, dstA_hbm, srcB_hbm, dstB_hbm, out_hbm,
          src_v, dst8, rows0, rows1, acc, sem0, sem1):
        c = lax.axis_index("c")
        s = lax.axis_index("s")

        zero = jnp.zeros((16,), jnp.float32)

        def zrow(r, _):
            for kk in range(F // 16):
                rows0[r, pl.ds(kk * 16, 16)] = zero
            return 0

        lax.fori_loop(0, CH, zrow, 0)

        def zstripe(j, _):
            pltpu.sync_copy(rows0, acc.at[pl.ds(s * STRIPE + j * CH, CH)])
            return 0

        lax.fori_loop(0, STRIPE // CH, zstripe, 0)
        plsc.subcore_barrier()

        def run(nch, dst_hbm):
            pltpu.async_copy(g_hbm.at[src_v.at[0]], rows0, sem0)

            def group(gi, _):
                pltpu.sync_copy(dst_hbm.at[s].at[pl.ds(gi * 8, 8)], dst8)
                for q in range(4):
                    j0 = gi * 8 + 2 * q
                    j1 = j0 + 1
                    pltpu.async_copy(g_hbm.at[src_v.at[j1]], rows1, sem1)
                    pltpu.make_async_copy(g_hbm.at[src_v.at[j0]], rows0,
                                          sem0).wait()
                    pltpu.sync_copy(rows0, acc.at[dst8.at[2 * q]], add=True)
                    pltpu.async_copy(g_hbm.at[src_v.at[j0 + 2]], rows0, sem0)
                    pltpu.make_async_copy(g_hbm.at[src_v.at[j1]], rows1,
                                          sem1).wait()
                    pltpu.sync_copy(rows1, acc.at[dst8.at[2 * q + 1]],
                                    add=True)
                return 0

            lax.fori_loop(0, nch // 8, group, 0)
            pltpu.make_async_copy(g_hbm.at[src_v.at[nch]], rows0, sem0).wait()

        @pl.when(c == 0)
        def _():
            pltpu.sync_copy(srcA_hbm.at[s], src_v.at[pl.ds(0, NCH0 + 1)])
            run(NCH0, dstA_hbm)

        @pl.when(c == 1)
        def _():
            pltpu.sync_copy(srcB_hbm.at[s], src_v)
            run(NCH1, dstB_hbm)

        plsc.subcore_barrier()

        def wb(j, _):
            off = s * STRIPE + j * 128
            pltpu.sync_copy(acc.at[pl.ds(off, 128)], out_hbm.at[c, pl.ds(off, 128)])
            return 0

        lax.fori_loop(0, STRIPE // 128, wb, 0)

    return k


def _make_sc_degree():
    @functools.partial(
        pl.kernel,
        mesh=_sc_mesh(),
        out_type=jax.ShapeDtypeStruct((2, NP), jnp.float32),
        scratch_types=[
            pltpu.VMEM((NCH1 + 1, CH), jnp.int32),
            pltpu.VMEM((128,), jnp.float32),
            pltpu.VMEM_SHARED((NP,), jnp.float32),
        ],
    )
    def k(dstA_hbm, dstB_hbm, out_hbm, dst_v, buf_v, acc):
        c = lax.axis_index("c")
        s = lax.axis_index("s")

        def fill(val):
            def go(r, _):
                buf_v[pl.ds(r * 16, 16)] = jnp.full((16,), val, jnp.float32)
                return 0
            lax.fori_loop(0, 8, go, 0)

        fill(0.0)

        def zstripe(j, _):
            pltpu.sync_copy(buf_v, acc.at[pl.ds(s * STRIPE + j * 128, 128)])
            return 0

        lax.fori_loop(0, STRIPE // 128, zstripe, 0)
        plsc.subcore_barrier()
        fill(1.0)

        def body(j, _):
            pltpu.sync_copy(buf_v.at[pl.ds(0, CH)], acc.at[dst_v.at[j]],
                            add=True)
            return 0

        @pl.when(c == 0)
        def _():
            pltpu.sync_copy(dstA_hbm.at[s], dst_v.at[pl.ds(0, NCH0 + 1)])
            lax.fori_loop(0, NCH0, body, 0)

        @pl.when(c == 1)
        def _():
            pltpu.sync_copy(dstB_hbm.at[s], dst_v)
            lax.fori_loop(0, NCH1, body, 0)

        plsc.subcore_barrier()

        def wb(j, _):
            off = s * STRIPE + j * 128
            pltpu.sync_copy(acc.at[pl.ds(off, 128)], out_hbm.at[c, pl.ds(off, 128)])
            return 0

        lax.fori_loop(0, STRIPE // 128, wb, 0)

    return k


_SC_CACHE = {}


def _sc_deg():
    if "deg" not in _SC_CACHE:
        _SC_CACHE["deg"] = _make_sc_degree()
    return _SC_CACHE["deg"]


def _sc_scat(F):
    if F not in _SC_CACHE:
        _SC_CACHE[F] = _make_sc_scatter(F)
    return _SC_CACHE[F]


BR = 1024
_GRID = NP // BR


def _row_spec(f=None):
    if f is None:
        return pl.BlockSpec((BR,), lambda i: (i,))
    return pl.BlockSpec((BR, f), lambda i: (i, 0))


def _full_spec(*shape):
    nd = len(shape)
    return pl.BlockSpec(shape, lambda i: (0,) * nd)


def _tc_prep_body(deg0, deg1, xr, dinv_o, g1_o):
    d = deg0[...] + deg1[...] + 1.0
    dinv = lax.rsqrt(d)
    dinv_o[...] = dinv
    g1_o[...] = xr[...] * dinv[:, None]


_tc_prep = pl.pallas_call(
    _tc_prep_body,
    grid=(_GRID,),
    in_specs=[_row_spec(), _row_spec(), _row_spec(128)],
    out_specs=[_row_spec(), _row_spec(128)],
    out_shape=[
        jax.ShapeDtypeStruct((NP,), jnp.float32),
        jax.ShapeDtypeStruct((NP, 128), jnp.float32),
    ],
)


def _tc_l1l2_body(s1a, s1b, g1, dinv, w1, b1, w2p, g2_o):
    dv = dinv[...][:, None]
    t = (s1a[...] + s1b[...] + g1[...]) * dv
    h = jnp.maximum(
        lax.dot_general(t, w1[...], (((1,), (1,)), ((), ())),
                        preferred_element_type=jnp.float32) + b1[...][None, :],
        0.0)
    g2_o[...] = lax.dot_general(h, w2p[...], (((1,), (1,)), ((), ())),
                                preferred_element_type=jnp.float32) * dv


_tc_l1l2 = pl.pallas_call(
    _tc_l1l2_body,
    grid=(_GRID,),
    in_specs=[_row_spec(128), _row_spec(128), _row_spec(128), _row_spec(),
              _full_spec(128, 128), _full_spec(128), _full_spec(128, 128)],
    out_specs=[_row_spec(128)],
    out_shape=[jax.ShapeDtypeStruct((NP, 128), jnp.float32)],
)


def _tc_l2fin_body(s2a, s2b, g2, dinv, b2, z_o, g3_o):
    dv = dinv[...][:, None]
    t = ((s2a[...] + s2b[...] + g2[...]) * dv)[:, :64]
    z = jnp.maximum(t + b2[...][None, :], 0.0)
    z_o[...] = z
    g3_o[...] = jnp.concatenate(
        [z * dv, jnp.zeros((BR, 64), jnp.float32)], axis=1)


_tc_l2fin = pl.pallas_call(
    _tc_l2fin_body,
    grid=(_GRID,),
    in_specs=[_row_spec(128), _row_spec(128), _row_spec(128), _row_spec(),
              _full_spec(64)],
    out_specs=[_row_spec(64), _row_spec(128)],
    out_shape=[
        jax.ShapeDtypeStruct((NP, 64), jnp.float32),
        jax.ShapeDtypeStruct((NP, 128), jnp.float32),
    ],
)


def _tc_l3l5_body(s3a, s3b, g3, dinv, w3, b3, w5, b5, w4, s_o, g4_o):
    dv = dinv[...][:, None]
    q = ((s3a[...] + s3b[...] + g3[...]) * dv)[:, :64]
    a = jnp.maximum(
        lax.dot_general(q, w3[...], (((1,), (1,)), ((), ())),
                        preferred_element_type=jnp.float32) + b3[...][None, :],
        0.0)
    s_o[...] = jnp.maximum(
        lax.dot_general(q, w5[...], (((1,), (1,)), ((), ())),
                        preferred_element_type=jnp.float32) + b5[...][None, :],
        0.0)
    g4_o[...] = lax.dot_general(a, w4[...], (((1,), (1,)), ((), ())),
                                preferred_element_type=jnp.float32) * dv


_tc_l3l5 = pl.pallas_call(
    _tc_l3l5_body,
    grid=(_GRID,),
    in_specs=[_row_spec(128), _row_spec(128), _row_spec(128), _row_spec(),
              _full_spec(128, 64), _full_spec(128),
              _full_spec(64, 64), _full_spec(64),
              _full_spec(128, 128)],
    out_specs=[_row_spec(64), _row_spec(128)],
    out_shape=[
        jax.ShapeDtypeStruct((NP, 64), jnp.float32),
        jax.ShapeDtypeStruct((NP, 128), jnp.float32),
    ],
)


def _tc_l4fin_body(s4a, s4b, g4, dinv, b4, xh_o):
    dv = dinv[...][:, None]
    xh_o[...] = (s4a[...] + s4b[...] + g4[...]) * dv + b4[...][None, :]


_tc_l4fin = pl.pallas_call(
    _tc_l4fin_body,
    grid=(_GRID,),
    in_specs=[_row_spec(128), _row_spec(128), _row_spec(128), _row_spec(),
              _full_spec(128)],
    out_specs=[_row_spec(128)],
    out_shape=[jax.ShapeDtypeStruct((NP, 128), jnp.float32)],
)

BA = 1024


def _tc_ahat_body(si, sj, o):
    o[...] = lax.dot_general(si[...], sj[...], (((1,), (1,)), ((), ())),
                             preferred_element_type=jnp.float32)


_tc_ahat = pl.pallas_call(
    _tc_ahat_body,
    grid=(pl.cdiv(N, BA), pl.cdiv(N, BA)),
    in_specs=[
        pl.BlockSpec((BA, 64), lambda i, j: (i, 0)),
        pl.BlockSpec((BA, 64), lambda i, j: (j, 0)),
    ],
    out_specs=pl.BlockSpec((BA, BA), lambda i, j: (i, j)),
    out_shape=jax.ShapeDtypeStruct((N, N), jnp.float32),
)


def kernel(x, edge_index, W1, b1, W2, b2, W3, b3, W4, b4, W5, b5):
    src = edge_index[0]
    dst = edge_index[1]
    npad = EP - E
    nA = 16 * NCH0 * CH
    src_p = jnp.concatenate([src, jnp.zeros((npad,), src.dtype)])
    dst_p = jnp.concatenate([dst, jnp.full((npad,), N, dst.dtype)])
    zrow_ = jnp.zeros((16, 1, CH), src.dtype)
    nch_ = jnp.full((16, 1, CH), N, dst.dtype)
    srcA = jnp.concatenate([src_p[:nA].reshape(16, NCH0, CH), zrow_], axis=1)
    dstA = jnp.concatenate([dst_p[:nA].reshape(16, NCH0, CH), nch_], axis=1)
    srcB = jnp.concatenate([src_p[nA:].reshape(16, NCH1, CH), zrow_], axis=1)
    dstB = jnp.concatenate([dst_p[nA:].reshape(16, NCH1, CH), nch_], axis=1)
    x_p = jnp.pad(x, ((0, NP - N), (0, 0)))
    W2p = jnp.pad(W2, ((0, 64), (0, 0)))

    deg = _sc_deg()(dstA, dstB)
    dinv, g1 = _tc_prep(deg[0], deg[1], x_p)

    scat = _sc_scat(128)
    s1 = scat(g1, srcA, dstA, srcB, dstB)
    (g2,) = _tc_l1l2(s1[0], s1[1], g1, dinv, W1, b1, W2p)

    s2 = scat(g2, srcA, dstA, srcB, dstB)
    z_full, g3 = _tc_l2fin(s2[0], s2[1], g2, dinv, b2)

    s3 = scat(g3, srcA, dstA, srcB, dstB)
    s_full, g4 = _tc_l3l5(s3[0], s3[1], g3, dinv, W3, b3, W5, b5, W4)

    s4 = scat(g4, srcA, dstA, srcB, dstB)
    (xh,) = _tc_l4fin(s4[0], s4[1], g4, dinv, b4)

    a_hat = _tc_ahat(s_full[:N], s_full[:N])
    return (xh[:N], a_hat, z_full[:N])


# speedup vs baseline: 1.2694x; 1.2694x over previous
import functools

import jax
import jax.numpy as jnp
from jax import lax
from jax.experimental import pallas as pl
from jax.experimental.pallas import tpu as pltpu
from jax.experimental.pallas import tpu_sc as plsc

N = 10000
NP = 10240
E = 320000
EP = 327680
NW = 32
NCH = EP // NW // 128
STRIPE = NP // 16


def _sc_mesh():
    return plsc.VectorSubcoreMesh(core_axis_name="c", subcore_axis_name="s")


def _make_sc_scatter(F):
    @functools.partial(
        pl.kernel,
        mesh=_sc_mesh(),
        out_type=jax.ShapeDtypeStruct((2, NP, F), jnp.float32),
        scratch_types=[
            pltpu.VMEM((NCH, 128), jnp.int32),
            pltpu.VMEM((NCH, 128), jnp.int32),
            pltpu.VMEM((128, F), jnp.float32),
            pltpu.VMEM_SHARED((NP, F), jnp.float32),
            pltpu.SemaphoreType.DMA,
        ],
    )
    def k(g_hbm, src_hbm, dst_hbm, out_hbm, src_v, dst_v, rows_v, acc, sem):
        c = lax.axis_index("c")
        s = lax.axis_index("s")
        w = s * 2 + c
        pltpu.sync_copy(src_hbm.at[w], src_v)
        pltpu.sync_copy(dst_hbm.at[w], dst_v)

        zero = jnp.zeros((16,), jnp.float32)

        def zrow(r, _):
            for kk in range(F // 16):
                rows_v[r, pl.ds(kk * 16, 16)] = zero
            return 0

        lax.fori_loop(0, 128, zrow, 0)

        def zstripe(j, _):
            pltpu.sync_copy(rows_v, acc.at[pl.ds(s * STRIPE + j * 128, 128)])
            return 0

        lax.fori_loop(0, STRIPE // 128, zstripe, 0)
        plsc.subcore_barrier()

        def body(j, _):
            pltpu.async_copy(g_hbm.at[src_v.at[j]], rows_v, sem).wait()
            pltpu.sync_copy(rows_v, acc.at[dst_v.at[j]], add=True)
            return 0

        lax.fori_loop(0, NCH, body, 0)
        plsc.subcore_barrier()

        def wb(j, _):
            off = s * STRIPE + j * 128
            pltpu.sync_copy(acc.at[pl.ds(off, 128)], out_hbm.at[c, pl.ds(off, 128)])
            return 0

        lax.fori_loop(0, STRIPE // 128, wb, 0)

    return k


def _make_sc_degree():
    @functools.partial(
        pl.kernel,
        mesh=_sc_mesh(),
        out_type=jax.ShapeDtypeStruct((2, NP), jnp.float32),
        scratch_types=[
            pltpu.VMEM((NCH, 128), jnp.int32),
            pltpu.VMEM((128,), jnp.float32),
            pltpu.VMEM_SHARED((NP,), jnp.float32),
        ],
    )
    def k(dst_hbm, out_hbm, dst_v, buf_v, acc):
        c = lax.axis_index("c")
        s = lax.axis_index("s")
        w = s * 2 + c
        pltpu.sync_copy(dst_hbm.at[w], dst_v)

        def fill(val):
            def go(r, _):
                buf_v[pl.ds(r * 16, 16)] = jnp.full((16,), val, jnp.float32)
                return 0
            lax.fori_loop(0, 8, go, 0)

        fill(0.0)

        def zstripe(j, _):
            pltpu.sync_copy(buf_v, acc.at[pl.ds(s * STRIPE + j * 128, 128)])
            return 0

        lax.fori_loop(0, STRIPE // 128, zstripe, 0)
        plsc.subcore_barrier()
        fill(1.0)

        def body(j, _):
            pltpu.sync_copy(buf_v, acc.at[dst_v.at[j]], add=True)
            return 0

        lax.fori_loop(0, NCH, body, 0)
        plsc.subcore_barrier()

        def wb(j, _):
            off = s * STRIPE + j * 128
            pltpu.sync_copy(acc.at[pl.ds(off, 128)], out_hbm.at[c, pl.ds(off, 128)])
            return 0

        lax.fori_loop(0, STRIPE // 128, wb, 0)

    return k


_SC_CACHE = {}


def _sc_deg():
    if "deg" not in _SC_CACHE:
        _SC_CACHE["deg"] = _make_sc_degree()
    return _SC_CACHE["deg"]


def _sc_scat(F):
    if F not in _SC_CACHE:
        _SC_CACHE[F] = _make_sc_scatter(F)
    return _SC_CACHE[F]


BR = 1024
_GRID = NP // BR


def _row_spec(f=None):
    if f is None:
        return pl.BlockSpec((BR,), lambda i: (i,))
    return pl.BlockSpec((BR, f), lambda i: (i, 0))


def _full_spec(*shape):
    nd = len(shape)
    return pl.BlockSpec(shape, lambda i: (0,) * nd)


def _tc_prep_body(deg0, deg1, xr, dinv_o, g1_o):
    d = deg0[...] + deg1[...] + 1.0
    dinv = lax.rsqrt(d)
    dinv_o[...] = dinv
    g1_o[...] = xr[...] * dinv[:, None]


_tc_prep = pl.pallas_call(
    _tc_prep_body,
    grid=(_GRID,),
    in_specs=[_row_spec(), _row_spec(), _row_spec(128)],
    out_specs=[_row_spec(), _row_spec(128)],
    out_shape=[
        jax.ShapeDtypeStruct((NP,), jnp.float32),
        jax.ShapeDtypeStruct((NP, 128), jnp.float32),
    ],
)


def _tc_l1l2_body(s1a, s1b, g1, dinv, w1, b1, w2p, g2_o):
    dv = dinv[...][:, None]
    t = (s1a[...] + s1b[...] + g1[...]) * dv
    h = jnp.maximum(
        lax.dot_general(t, w1[...], (((1,), (1,)), ((), ())),
                        preferred_element_type=jnp.float32) + b1[...][None, :],
        0.0)
    g2_o[...] = lax.dot_general(h, w2p[...], (((1,), (1,)), ((), ())),
                                preferred_element_type=jnp.float32) * dv


_tc_l1l2 = pl.pallas_call(
    _tc_l1l2_body,
    grid=(_GRID,),
    in_specs=[_row_spec(128), _row_spec(128), _row_spec(128), _row_spec(),
              _full_spec(128, 128), _full_spec(128), _full_spec(128, 128)],
    out_specs=[_row_spec(128)],
    out_shape=[jax.ShapeDtypeStruct((NP, 128), jnp.float32)],
)


def _tc_l2fin_body(s2a, s2b, g2, dinv, b2, z_o, g3_o):
    dv = dinv[...][:, None]
    t = ((s2a[...] + s2b[...] + g2[...]) * dv)[:, :64]
    z = jnp.maximum(t + b2[...][None, :], 0.0)
    z_o[...] = z
    g3_o[...] = jnp.concatenate(
        [z * dv, jnp.zeros((BR, 64), jnp.float32)], axis=1)


_tc_l2fin = pl.pallas_call(
    _tc_l2fin_body,
    grid=(_GRID,),
    in_specs=[_row_spec(128), _row_spec(128), _row_spec(128), _row_spec(),
              _full_spec(64)],
    out_specs=[_row_spec(64), _row_spec(128)],
    out_shape=[
        jax.ShapeDtypeStruct((NP, 64), jnp.float32),
        jax.ShapeDtypeStruct((NP, 128), jnp.float32),
    ],
)


def _tc_l3l5_body(s3a, s3b, g3, dinv, w3, b3, w5, b5, w4, s_o, g4_o):
    dv = dinv[...][:, None]
    q = ((s3a[...] + s3b[...] + g3[...]) * dv)[:, :64]
    a = jnp.maximum(
        lax.dot_general(q, w3[...], (((1,), (1,)), ((), ())),
                        preferred_element_type=jnp.float32) + b3[...][None, :],
        0.0)
    s_o[...] = jnp.maximum(
        lax.dot_general(q, w5[...], (((1,), (1,)), ((), ())),
                        preferred_element_type=jnp.float32) + b5[...][None, :],
        0.0)
    g4_o[...] = lax.dot_general(a, w4[...], (((1,), (1,)), ((), ())),
                                preferred_element_type=jnp.float32) * dv


_tc_l3l5 = pl.pallas_call(
    _tc_l3l5_body,
    grid=(_GRID,),
    in_specs=[_row_spec(128), _row_spec(128), _row_spec(128), _row_spec(),
              _full_spec(128, 64), _full_spec(128),
              _full_spec(64, 64), _full_spec(64),
              _full_spec(128, 128)],
    out_specs=[_row_spec(64), _row_spec(128)],
    out_shape=[
        jax.ShapeDtypeStruct((NP, 64), jnp.float32),
        jax.ShapeDtypeStruct((NP, 128), jnp.float32),
    ],
)


def _tc_l4fin_body(s4a, s4b, g4, dinv, b4, xh_o):
    dv = dinv[...][:, None]
    xh_o[...] = (s4a[...] + s4b[...] + g4[...]) * dv + b4[...][None, :]


_tc_l4fin = pl.pallas_call(
    _tc_l4fin_body,
    grid=(_GRID,),
    in_specs=[_row_spec(128), _row_spec(128), _row_spec(128), _row_spec(),
              _full_spec(128)],
    out_specs=[_row_spec(128)],
    out_shape=[jax.ShapeDtypeStruct((NP, 128), jnp.float32)],
)

BA = 1024


def _tc_ahat_body(si, sj, o):
    o[...] = lax.dot_general(si[...], sj[...], (((1,), (1,)), ((), ())),
                             preferred_element_type=jnp.float32)


_tc_ahat = pl.pallas_call(
    _tc_ahat_body,
    grid=(pl.cdiv(N, BA), pl.cdiv(N, BA)),
    in_specs=[
        pl.BlockSpec((BA, 64), lambda i, j: (i, 0)),
        pl.BlockSpec((BA, 64), lambda i, j: (j, 0)),
    ],
    out_specs=pl.BlockSpec((BA, BA), lambda i, j: (i, j)),
    out_shape=jax.ShapeDtypeStruct((N, N), jnp.float32),
)


def kernel(x, edge_index, W1, b1, W2, b2, W3, b3, W4, b4, W5, b5):
    src = edge_index[0]
    dst = edge_index[1]
    npad = EP - E
    src_p = jnp.concatenate(
        [src, jnp.zeros((npad,), src.dtype)]).reshape(NW, NCH, 128)
    dst_p = jnp.concatenate(
        [dst, jnp.full((npad,), N, dst.dtype)]).reshape(NW, NCH, 128)
    x_p = jnp.pad(x, ((0, NP - N), (0, 0)))
    W2p = jnp.pad(W2, ((0, 64), (0, 0)))

    deg = _sc_deg()(dst_p)
    dinv, g1 = _tc_prep(deg[0], deg[1], x_p)

    s1 = _sc_scat(128)(g1, src_p, dst_p)
    (g2,) = _tc_l1l2(s1[0], s1[1], g1, dinv, W1, b1, W2p)

    s2 = _sc_scat(128)(g2, src_p, dst_p)
    z_full, g3 = _tc_l2fin(s2[0], s2[1], g2, dinv, b2)

    s3 = _sc_scat(128)(g3, src_p, dst_p)
    s_full, g4 = _tc_l3l5(s3[0], s3[1], g3, dinv, W3, b3, W5, b5, W4)

    s4 = _sc_scat(128)(g4, src_p, dst_p)
    (xh,) = _tc_l4fin(s4[0], s4[1], g4, dinv, b4)

    a_hat = _tc_ahat(s_full[:N], s_full[:N])
    return (xh[:N], a_hat, z_full[:N])


# speedup vs baseline: 1.3602x; 1.0715x over previous
import functools

import jax
import jax.numpy as jnp
from jax import lax
from jax.experimental import pallas as pl
from jax.experimental.pallas import tpu as pltpu
from jax.experimental.pallas import tpu_sc as plsc

N = 10000
NP = 10240
E = 320000
EP = 327680
NCH0 = 120
NCH1 = 40
NCHMX = max(NCH0, NCH1)
STRIPE = NP // 16


def _sc_mesh():
    return plsc.VectorSubcoreMesh(core_axis_name="c", subcore_axis_name="s")


def _make_sc_scatter(F):
    @functools.partial(
        pl.kernel,
        mesh=_sc_mesh(),
        out_type=jax.ShapeDtypeStruct((2, NP, F), jnp.float32),
        scratch_types=[
            pltpu.VMEM((NCHMX, 128), jnp.int32),
            pltpu.VMEM((NCHMX, 128), jnp.int32),
            pltpu.VMEM((128, F), jnp.float32),
            pltpu.VMEM_SHARED((NP, F), jnp.float32),
            pltpu.SemaphoreType.DMA,
        ],
    )
    def k(g_hbm, srcA_hbm, dstA_hbm, srcB_hbm, dstB_hbm, out_hbm,
          src_v, dst_v, rows_v, acc, sem):
        c = lax.axis_index("c")
        s = lax.axis_index("s")

        zero = jnp.zeros((16,), jnp.float32)

        def zrow(r, _):
            for kk in range(F // 16):
                rows_v[r, pl.ds(kk * 16, 16)] = zero
            return 0

        lax.fori_loop(0, 128, zrow, 0)

        def zstripe(j, _):
            pltpu.sync_copy(rows_v, acc.at[pl.ds(s * STRIPE + j * 128, 128)])
            return 0

        lax.fori_loop(0, STRIPE // 128, zstripe, 0)
        plsc.subcore_barrier()

        def body(j, _):
            pltpu.async_copy(g_hbm.at[src_v.at[j]], rows_v, sem).wait()
            pltpu.sync_copy(rows_v, acc.at[dst_v.at[j]], add=True)
            return 0

        @pl.when(c == 0)
        def _():
            pltpu.sync_copy(srcA_hbm.at[s], src_v)
            pltpu.sync_copy(dstA_hbm.at[s], dst_v)
            lax.fori_loop(0, NCH0, body, 0)

        @pl.when(c == 1)
        def _():
            pltpu.sync_copy(srcB_hbm.at[s], src_v.at[pl.ds(0, NCH1)])
            pltpu.sync_copy(dstB_hbm.at[s], dst_v.at[pl.ds(0, NCH1)])
            lax.fori_loop(0, NCH1, body, 0)

        plsc.subcore_barrier()

        def wb(j, _):
            off = s * STRIPE + j * 128
            pltpu.sync_copy(acc.at[pl.ds(off, 128)], out_hbm.at[c, pl.ds(off, 128)])
            return 0

        lax.fori_loop(0, STRIPE // 128, wb, 0)

    return k


def _make_sc_degree():
    @functools.partial(
        pl.kernel,
        mesh=_sc_mesh(),
        out_type=jax.ShapeDtypeStruct((2, NP), jnp.float32),
        scratch_types=[
            pltpu.VMEM((NCHMX, 128), jnp.int32),
            pltpu.VMEM((128,), jnp.float32),
            pltpu.VMEM_SHARED((NP,), jnp.float32),
        ],
    )
    def k(dstA_hbm, dstB_hbm, out_hbm, dst_v, buf_v, acc):
        c = lax.axis_index("c")
        s = lax.axis_index("s")

        def fill(val):
            def go(r, _):
                buf_v[pl.ds(r * 16, 16)] = jnp.full((16,), val, jnp.float32)
                return 0
            lax.fori_loop(0, 8, go, 0)

        fill(0.0)

        def zstripe(j, _):
            pltpu.sync_copy(buf_v, acc.at[pl.ds(s * STRIPE + j * 128, 128)])
            return 0

        lax.fori_loop(0, STRIPE // 128, zstripe, 0)
        plsc.subcore_barrier()
        fill(1.0)

        def body(j, _):
            pltpu.sync_copy(buf_v, acc.at[dst_v.at[j]], add=True)
            return 0

        @pl.when(c == 0)
        def _():
            pltpu.sync_copy(dstA_hbm.at[s], dst_v)
            lax.fori_loop(0, NCH0, body, 0)

        @pl.when(c == 1)
        def _():
            pltpu.sync_copy(dstB_hbm.at[s], dst_v.at[pl.ds(0, NCH1)])
            lax.fori_loop(0, NCH1, body, 0)

        plsc.subcore_barrier()

        def wb(j, _):
            off = s * STRIPE + j * 128
            pltpu.sync_copy(acc.at[pl.ds(off, 128)], out_hbm.at[c, pl.ds(off, 128)])
            return 0

        lax.fori_loop(0, STRIPE // 128, wb, 0)

    return k


_SC_CACHE = {}


def _sc_deg():
    if "deg" not in _SC_CACHE:
        _SC_CACHE["deg"] = _make_sc_degree()
    return _SC_CACHE["deg"]


def _sc_scat(F):
    if F not in _SC_CACHE:
        _SC_CACHE[F] = _make_sc_scatter(F)
    return _SC_CACHE[F]


BR = 1024
_GRID = NP // BR


def _row_spec(f=None):
    if f is None:
        return pl.BlockSpec((BR,), lambda i: (i,))
    return pl.BlockSpec((BR, f), lambda i: (i, 0))


def _full_spec(*shape):
    nd = len(shape)
    return pl.BlockSpec(shape, lambda i: (0,) * nd)


def _tc_prep_body(deg0, deg1, xr, dinv_o, g1_o):
    d = deg0[...] + deg1[...] + 1.0
    dinv = lax.rsqrt(d)
    dinv_o[...] = dinv
    g1_o[...] = xr[...] * dinv[:, None]


_tc_prep = pl.pallas_call(
    _tc_prep_body,
    grid=(_GRID,),
    in_specs=[_row_spec(), _row_spec(), _row_spec(128)],
    out_specs=[_row_spec(), _row_spec(128)],
    out_shape=[
        jax.ShapeDtypeStruct((NP,), jnp.float32),
        jax.ShapeDtypeStruct((NP, 128), jnp.float32),
    ],
)


def _tc_l1l2_body(s1a, s1b, g1, dinv, w1, b1, w2p, g2_o):
    dv = dinv[...][:, None]
    t = (s1a[...] + s1b[...] + g1[...]) * dv
    h = jnp.maximum(
        lax.dot_general(t, w1[...], (((1,), (1,)), ((), ())),
                        preferred_element_type=jnp.float32) + b1[...][None, :],
        0.0)
    g2_o[...] = lax.dot_general(h, w2p[...], (((1,), (1,)), ((), ())),
                                preferred_element_type=jnp.float32) * dv


_tc_l1l2 = pl.pallas_call(
    _tc_l1l2_body,
    grid=(_GRID,),
    in_specs=[_row_spec(128), _row_spec(128), _row_spec(128), _row_spec(),
              _full_spec(128, 128), _full_spec(128), _full_spec(128, 128)],
    out_specs=[_row_spec(128)],
    out_shape=[jax.ShapeDtypeStruct((NP, 128), jnp.float32)],
)


def _tc_l2fin_body(s2a, s2b, g2, dinv, b2, z_o, g3_o):
    dv = dinv[...][:, None]
    t = ((s2a[...] + s2b[...] + g2[...]) * dv)[:, :64]
    z = jnp.maximum(t + b2[...][None, :], 0.0)
    z_o[...] = z
    g3_o[...] = jnp.concatenate(
        [z * dv, jnp.zeros((BR, 64), jnp.float32)], axis=1)


_tc_l2fin = pl.pallas_call(
    _tc_l2fin_body,
    grid=(_GRID,),
    in_specs=[_row_spec(128), _row_spec(128), _row_spec(128), _row_spec(),
              _full_spec(64)],
    out_specs=[_row_spec(64), _row_spec(128)],
    out_shape=[
        jax.ShapeDtypeStruct((NP, 64), jnp.float32),
        jax.ShapeDtypeStruct((NP, 128), jnp.float32),
    ],
)


def _tc_l3l5_body(s3a, s3b, g3, dinv, w3, b3, w5, b5, w4, s_o, g4_o):
    dv = dinv[...][:, None]
    q = ((s3a[...] + s3b[...] + g3[...]) * dv)[:, :64]
    a = jnp.maximum(
        lax.dot_general(q, w3[...], (((1,), (1,)), ((), ())),
                        preferred_element_type=jnp.float32) + b3[...][None, :],
        0.0)
    s_o[...] = jnp.maximum(
        lax.dot_general(q, w5[...], (((1,), (1,)), ((), ())),
                        preferred_element_type=jnp.float32) + b5[...][None, :],
        0.0)
    g4_o[...] = lax.dot_general(a, w4[...], (((1,), (1,)), ((), ())),
                                preferred_element_type=jnp.float32) * dv


_tc_l3l5 = pl.pallas_call(
    _tc_l3l5_body,
    grid=(_GRID,),
    in_specs=[_row_spec(128), _row_spec(128), _row_spec(128), _row_spec(),
              _full_spec(128, 64), _full_spec(128),
              _full_spec(64, 64), _full_spec(64),
              _full_spec(128, 128)],
    out_specs=[_row_spec(64), _row_spec(128)],
    out_shape=[
        jax.ShapeDtypeStruct((NP, 64), jnp.float32),
        jax.ShapeDtypeStruct((NP, 128), jnp.float32),
    ],
)


def _tc_l4fin_body(s4a, s4b, g4, dinv, b4, xh_o):
    dv = dinv[...][:, None]
    xh_o[...] = (s4a[...] + s4b[...] + g4[...]) * dv + b4[...][None, :]


_tc_l4fin = pl.pallas_call(
    _tc_l4fin_body,
    grid=(_GRID,),
    in_specs=[_row_spec(128), _row_spec(128), _row_spec(128), _row_spec(),
              _full_spec(128)],
    out_specs=[_row_spec(128)],
    out_shape=[jax.ShapeDtypeStruct((NP, 128), jnp.float32)],
)

BA = 1024


def _tc_ahat_body(si, sj, o):
    o[...] = lax.dot_general(si[...], sj[...], (((1,), (1,)), ((), ())),
                             preferred_element_type=jnp.float32)


_tc_ahat = pl.pallas_call(
    _tc_ahat_body,
    grid=(pl.cdiv(N, BA), pl.cdiv(N, BA)),
    in_specs=[
        pl.BlockSpec((BA, 64), lambda i, j: (i, 0)),
        pl.BlockSpec((BA, 64), lambda i, j: (j, 0)),
    ],
    out_specs=pl.BlockSpec((BA, BA), lambda i, j: (i, j)),
    out_shape=jax.ShapeDtypeStruct((N, N), jnp.float32),
)


def kernel(x, edge_index, W1, b1, W2, b2, W3, b3, W4, b4, W5, b5):
    src = edge_index[0]
    dst = edge_index[1]
    npad = EP - E
    nA = 16 * NCH0 * 128
    src_p = jnp.concatenate([src, jnp.zeros((npad,), src.dtype)])
    dst_p = jnp.concatenate([dst, jnp.full((npad,), N, dst.dtype)])
    srcA = src_p[:nA].reshape(16, NCH0, 128)
    dstA = dst_p[:nA].reshape(16, NCH0, 128)
    srcB = src_p[nA:].reshape(16, NCH1, 128)
    dstB = dst_p[nA:].reshape(16, NCH1, 128)
    x_p = jnp.pad(x, ((0, NP - N), (0, 0)))
    W2p = jnp.pad(W2, ((0, 64), (0, 0)))

    deg = _sc_deg()(dstA, dstB)
    dinv, g1 = _tc_prep(deg[0], deg[1], x_p)

    scat = _sc_scat(128)
    s1 = scat(g1, srcA, dstA, srcB, dstB)
    (g2,) = _tc_l1l2(s1[0], s1[1], g1, dinv, W1, b1, W2p)

    s2 = scat(g2, srcA, dstA, srcB, dstB)
    z_full, g3 = _tc_l2fin(s2[0], s2[1], g2, dinv, b2)

    s3 = scat(g3, srcA, dstA, srcB, dstB)
    s_full, g4 = _tc_l3l5(s3[0], s3[1], g3, dinv, W3, b3, W5, b5, W4)

    s4 = scat(g4, srcA, dstA, srcB, dstB)
    (xh,) = _tc_l4fin(s4[0], s4[1], g4, dinv, b4)

    a_hat = _tc_ahat(s_full[:N], s_full[:N])
    return (xh[:N], a_hat, z_full[:N])
